# baseline (speedup 1.0000x reference)
import jax
import jax.numpy as jnp
from jax.experimental import pallas as pl

_BN = 8192


def _body(a_ref, b_ref, xui_ref, a_out_ref):
    av = a_ref[...]
    a_out_ref[...] = av
    xui_ref[...] = jnp.sum(av * b_ref[...], axis=0)


def kernel(gum, gim):
    n_rows, n_cols = gum.shape
    a = gum.T
    b = gim.T
    grid = (n_rows // _BN,)
    xui, a_o = pl.pallas_call(
        _body,
        grid=grid,
        in_specs=[
            pl.BlockSpec((n_cols, _BN), lambda i: (0, i)),
            pl.BlockSpec((n_cols, _BN), lambda i: (0, i)),
        ],
        out_specs=[
            pl.BlockSpec((_BN,), lambda i: (i,)),
            pl.BlockSpec((n_cols, _BN), lambda i: (0, i)),
        ],
        out_shape=[
            jax.ShapeDtypeStruct((n_rows,), jnp.float32),
            jax.ShapeDtypeStruct((n_cols, n_rows), jnp.float32),
        ],
    )(a, b)
    return (xui, a_o.T, gim)


# speedup vs baseline: 1.0419x; 1.0419x over previous
import jax
import jax.numpy as jnp
from jax.experimental import pallas as pl

_BR = 16


def _body(a_ref, b_ref, xui_ref, a_out_ref, b_out_ref):
    av = a_ref[...]
    bv = b_ref[...]
    a_out_ref[...] = av
    b_out_ref[...] = bv
    part = jnp.sum(av * bv, axis=0)

    @pl.when(pl.program_id(0) == 0)
    def _init():
        xui_ref[...] = part

    @pl.when(pl.program_id(0) != 0)
    def _acc():
        xui_ref[...] += part


def kernel(gum, gim):
    n_rows, n_cols = gum.shape
    a = gum.T
    b = gim.T
    grid = (n_cols // _BR,)
    xui, a_o, b_o = pl.pallas_call(
        _body,
        grid=grid,
        in_specs=[
            pl.BlockSpec((_BR, n_rows), lambda i: (i, 0)),
            pl.BlockSpec((_BR, n_rows), lambda i: (i, 0)),
        ],
        out_specs=[
            pl.BlockSpec((n_rows,), lambda i: (0,)),
            pl.BlockSpec((_BR, n_rows), lambda i: (i, 0)),
            pl.BlockSpec((_BR, n_rows), lambda i: (i, 0)),
        ],
        out_shape=[
            jax.ShapeDtypeStruct((n_rows,), jnp.float32),
            jax.ShapeDtypeStruct((n_cols, n_rows), jnp.float32),
            jax.ShapeDtypeStruct((n_cols, n_rows), jnp.float32),
        ],
    )(a, b)
    return (xui, a_o.T, b_o.T)


# speedup vs baseline: 1.2369x; 1.1871x over previous
import jax
import jax.numpy as jnp
from jax.experimental import pallas as pl

_BR = 32


def _body(a_ref, b_ref, xui_ref, a_out_ref, b_out_ref):
    av = a_ref[...]
    bv = b_ref[...]
    a_out_ref[...] = av
    b_out_ref[...] = bv
    part = jnp.sum(av * bv, axis=0)

    @pl.when(pl.program_id(0) == 0)
    def _init():
        xui_ref[...] = part

    @pl.when(pl.program_id(0) != 0)
    def _acc():
        xui_ref[...] += part


def kernel(gum, gim):
    n_rows, n_cols = gum.shape
    a = gum.T
    b = gim.T
    grid = (n_cols // _BR,)
    xui, a_o, b_o = pl.pallas_call(
        _body,
        grid=grid,
        in_specs=[
            pl.BlockSpec((_BR, n_rows), lambda i: (i, 0)),
            pl.BlockSpec((_BR, n_rows), lambda i: (i, 0)),
        ],
        out_specs=[
            pl.BlockSpec((n_rows,), lambda i: (0,)),
            pl.BlockSpec((_BR, n_rows), lambda i: (i, 0)),
            pl.BlockSpec((_BR, n_rows), lambda i: (i, 0)),
        ],
        out_shape=[
            jax.ShapeDtypeStruct((n_rows,), jnp.float32),
            jax.ShapeDtypeStruct((n_cols, n_rows), jnp.float32),
            jax.ShapeDtypeStruct((n_cols, n_rows), jnp.float32),
        ],
    )(a, b)
    return (xui, a_o.T, b_o.T)


# speedup vs baseline: 1.2869x; 1.0404x over previous
import jax
import jax.numpy as jnp
from jax.experimental import pallas as pl

_BN = 8192


def _body(a_ref, b_ref, xui_ref, a_out_ref, b_out_ref):
    av = a_ref[...]
    bv = b_ref[...]
    a_out_ref[...] = av
    b_out_ref[...] = bv
    xui_ref[...] = jnp.sum(av * bv, axis=0)


def kernel(gum, gim):
    n_rows, n_cols = gum.shape
    a = gum.T
    b = gim.T
    grid = (n_rows // _BN,)
    xui, a_o, b_o = pl.pallas_call(
        _body,
        grid=grid,
        in_specs=[
            pl.BlockSpec((n_cols, _BN), lambda i: (0, i)),
            pl.BlockSpec((n_cols, _BN), lambda i: (0, i)),
        ],
        out_specs=[
            pl.BlockSpec((_BN,), lambda i: (i,)),
            pl.BlockSpec((n_cols, _BN), lambda i: (0, i)),
            pl.BlockSpec((n_cols, _BN), lambda i: (0, i)),
        ],
        out_shape=[
            jax.ShapeDtypeStruct((n_rows,), jnp.float32),
            jax.ShapeDtypeStruct((n_cols, n_rows), jnp.float32),
            jax.ShapeDtypeStruct((n_cols, n_rows), jnp.float32),
        ],
    )(a, b)
    return (xui, a_o.T, b_o.T)
